# baseline (device time: 42963 ns/iter reference)
import jax
import jax.numpy as jnp
from jax import lax
from jax.experimental import pallas as pl
from jax.experimental.pallas import tpu as pltpu

N_DEV = 4
SQ = 1024
SKV = 1024
HQ = 8
DH = 128
D = HQ * DH
CH = 256
HALF = CH // 2
SCALE = 0.08838834764831843
W = D + DH
BF16 = jnp.bfloat16


def kernel(x, Wq, K_ext, V_ext, Wo):
    x2 = x.reshape(SQ, D)
    K2 = K_ext.reshape(SKV, D)
    V2 = V_ext.reshape(SKV, D)

    def body(x_ref, wq_ref, k_ref, v_ref, wo_ref, out_ref,
             xp_ref, kr_ref, vr_ref, wqb_ref, wob_ref, part_ref,
             rsr_ref, rsl_ref, og_ref, rs_sems, ag_sems):
        my = lax.axis_index("i")
        left = (my - 1) % N_DEV
        right = (my + 1) % N_DEV

        barrier = pltpu.get_barrier_semaphore()
        for nbr in (left, right):
            pl.semaphore_signal(barrier, inc=1, device_id=(nbr,),
                                device_id_type=pl.DeviceIdType.MESH)
        pl.semaphore_wait(barrier, 2)

        wqb_ref[...] = wq_ref[...].astype(BF16)
        wob_ref[...] = wo_ref[...].astype(BF16)

        for r in range(4):
            for u in range(4):
                xp_ref[256 * r + 64 * u:256 * r + 64 * u + 64, :] = (
                    x_ref[64 * (4 * u + r):64 * (4 * u + r) + 64,
                          :].astype(BF16))

        def gather_kv(k):
            c = (my + k) % N_DEV
            for u in range(4):
                kv0 = 256 * u + 64 * c
                kr_ref[k, 64 * u:64 * u + 64, :] = (
                    k_ref[pl.ds(kv0, 64), :].astype(BF16))
                vr_ref[k, 64 * u:64 * u + 64, :] = (
                    v_ref[pl.ds(kv0, 64), :].astype(BF16))

        def compute_half(k, up):
            c = (my + k) % N_DEV
            off = 0 if up else HALF
            rows = slice(off, off + HALF)
            q = jnp.dot(xp_ref[pl.ds(CH * c + off, HALF), :], wqb_ref[...],
                        preferred_element_type=jnp.float32).astype(BF16)
            for h in range(HQ):
                c0, c1 = h * DH, (h + 1) * DH
                s = lax.dot_general(q[:, c0:c1], kr_ref[k, :, c0:c1],
                                    (((1,), (1,)), ((), ())),
                                    preferred_element_type=jnp.float32)
                w = jnp.exp(s * SCALE)
                part_ref[k, rows, c0:c1] = jnp.dot(
                    w.astype(BF16), vr_ref[k, :, c0:c1],
                    preferred_element_type=jnp.float32).astype(BF16)
                part_ref[k, rows, D + h:D + h + 1] = jnp.sum(
                    w, axis=1, keepdims=True).astype(BF16)

        UP = pl.ds(0, HALF)
        LO = pl.ds(HALF, HALF)

        RS_ADD_R = (2, 1, 0)
        RS_ADD_L = (2, 3, 0)

        def start_rs(dirn, t, src):
            to = right if dirn == 0 else left
            dst = rsr_ref if dirn == 0 else rsl_ref
            rd = pltpu.make_async_remote_copy(
                src_ref=src, dst_ref=dst.at[t],
                send_sem=rs_sems.at[2 * dirn, t],
                recv_sem=rs_sems.at[2 * dirn + 1, t],
                device_id=(to,), device_id_type=pl.DeviceIdType.MESH)
            rd.start()
            return rd

        pend = []

        gather_kv(3)
        compute_half(3, True)
        r0r = start_rs(0, 0, part_ref.at[3, UP])
        gather_kv(1)
        compute_half(1, False)
        r0l = start_rs(1, 0, part_ref.at[1, LO])
        pend += [r0r, r0l]

        gather_kv(2)
        compute_half(2, True)
        compute_half(2, False)

        r0r.wait_recv()
        rsr_ref[0, :, :] = rsr_ref[0] + part_ref[RS_ADD_R[0], 0:HALF, :]
        r1r = start_rs(0, 1, rsr_ref.at[0])
        r0l.wait_recv()
        rsl_ref[0, :, :] = rsl_ref[0] + part_ref[RS_ADD_L[0], HALF:CH, :]
        r1l = start_rs(1, 1, rsl_ref.at[0])
        pend += [r1r, r1l]

        compute_half(1, True)
        compute_half(3, False)

        r1r.wait_recv()
        rsr_ref[1, :, :] = rsr_ref[1] + part_ref[RS_ADD_R[1], 0:HALF, :]
        r2r = start_rs(0, 2, rsr_ref.at[1])
        r1l.wait_recv()
        rsl_ref[1, :, :] = rsl_ref[1] + part_ref[RS_ADD_L[1], HALF:CH, :]
        r2l = start_rs(1, 2, rsl_ref.at[1])
        pend += [r2r, r2l]

        gather_kv(0)
        compute_half(0, True)
        compute_half(0, False)

        r2r.wait_recv()
        rsr_ref[2, :, :] = rsr_ref[2] + part_ref[RS_ADD_R[2], 0:HALF, :]
        r2l.wait_recv()
        rsl_ref[2, :, :] = rsl_ref[2] + part_ref[RS_ADD_L[2], HALF:CH, :]

        def ag_send(idx, src, dst, to):
            rd = pltpu.make_async_remote_copy(
                src_ref=src, dst_ref=dst,
                send_sem=ag_sems.at[0, idx], recv_sem=ag_sems.at[1, idx],
                device_id=(to,), device_id_type=pl.DeviceIdType.MESH)
            rd.start()
            return rd

        def norm_half(row_sel, src):
            for h in range(HQ):
                c0, c1 = h * DH, (h + 1) * DH
                kr_ref[0, row_sel, c0:c1] = (
                    src[:, c0:c1].astype(jnp.float32)
                    / src[:, D + h:D + h + 1].astype(jnp.float32)
                ).astype(BF16)

        norm_half(slice(0, HALF), rsr_ref[2])
        og_ref[0, 0:HALF, :] = jnp.dot(
            kr_ref[0, 0:HALF, :], wob_ref[...],
            preferred_element_type=jnp.float32).astype(BF16)
        a1ru = ag_send(0, og_ref.at[0, UP], og_ref.at[3, UP], right)
        a1lu = ag_send(1, og_ref.at[0, UP], og_ref.at[1, UP], left)

        norm_half(slice(HALF, CH), rsl_ref[2])
        og_ref[0, HALF:CH, :] = jnp.dot(
            kr_ref[0, HALF:CH, :], wob_ref[...],
            preferred_element_type=jnp.float32).astype(BF16)
        a1rl = ag_send(2, og_ref.at[0, LO], og_ref.at[3, LO], right)
        a1ll = ag_send(3, og_ref.at[0, LO], og_ref.at[1, LO], left)
        pend += [a1ru, a1lu, a1rl, a1ll]

        def store_slot(k_slot, us):
            c = (my + k_slot) % N_DEV
            for u in us:
                out_ref[pl.ds(256 * u + 64 * c, 64), :] = (
                    og_ref[k_slot, 64 * u:64 * u + 64, :].astype(jnp.float32))

        store_slot(0, (0, 1, 2, 3))

        a1ru.wait_recv()
        a1lu.wait_recv()
        a2l = ag_send(4, og_ref.at[1, UP], og_ref.at[2, UP], left)
        a1rl.wait_recv()
        a1ll.wait_recv()
        a2r = ag_send(5, og_ref.at[3, LO], og_ref.at[2, LO], right)
        pend += [a2l, a2r]

        store_slot(1, (0, 1, 2, 3))
        store_slot(3, (0, 1, 2, 3))

        a2l.wait_recv()
        a2r.wait_recv()
        store_slot(2, (0, 1, 2, 3))

        for d in pend:
            d.wait_send()

    out = pl.pallas_call(
        body,
        out_shape=jax.ShapeDtypeStruct((SQ, D), jnp.float32),
        in_specs=[pl.BlockSpec(memory_space=pltpu.VMEM)] * 5,
        out_specs=pl.BlockSpec(memory_space=pltpu.VMEM),
        scratch_shapes=[
            pltpu.VMEM((SQ, D), BF16),
            pltpu.VMEM((4, CH, D), BF16),
            pltpu.VMEM((4, CH, D), BF16),
            pltpu.VMEM((D, D), BF16),
            pltpu.VMEM((D, D), BF16),
            pltpu.VMEM((4, CH, W), BF16),
            pltpu.VMEM((3, HALF, W), BF16),
            pltpu.VMEM((3, HALF, W), BF16),
            pltpu.VMEM((4, CH, D), BF16),
            pltpu.SemaphoreType.DMA((4, 3)),
            pltpu.SemaphoreType.DMA((2, 6)),
        ],
        compiler_params=pltpu.CompilerParams(collective_id=0),
    )(x2, Wq, K2, V2, Wo)
    return out.reshape(1, SQ, D)
